# baseline (speedup 1.0000x reference)
import functools

import jax
import jax.numpy as jnp
from jax import lax
from jax.experimental import pallas as pl
from jax.experimental.pallas import tpu as pltpu
from jax.experimental.pallas import tpu_sc as plsc

_B = 4096
_L = 50
_E = 200
_EP = 256
_HID = 256
_NCLS = 20

_NC = 2
_NS = 16
_NW = _NC * _NS
_BPW = _B // _NW
_K = 128
_NCHUNK = (_BPW * _L) // _K

_RV = 2048


def _pack_body(tab_ref, out_ref):
    out_ref[:, 0:_E] = tab_ref[...]
    out_ref[:, _E:_EP] = jnp.zeros((_RV, _EP - _E), jnp.float32)


@jax.jit
def _pack(emb_word, emb_bigram, emb_trigram):
    outs = []
    for tab in (emb_word, emb_bigram, emb_trigram):
        v = tab.shape[0]
        grid = (pl.cdiv(v, _RV),)
        outs.append(pl.pallas_call(
            _pack_body,
            grid=grid,
            in_specs=[pl.BlockSpec((_RV, _E), lambda i: (i, 0))],
            out_specs=pl.BlockSpec((_RV, _EP), lambda i: (i, 0)),
            out_shape=jax.ShapeDtypeStruct((v, _EP), jnp.float32),
        )(tab))
    return outs


_EA = 128


def _sc_pool_body(x0r, x2r, x3r, dest_hbm, zeros_hbm,
                  pkw, pkb, pkt,
                  owa, owb, oba, obb, ota, otb,
                  idx_v, dest_v, bufa, bufb, acca, accb, sema, semb):
    c = lax.axis_index("c")
    s = lax.axis_index("s")
    w = c * _NS + s
    base = w * _BPW

    pltpu.sync_copy(dest_hbm.at[s], dest_v)

    tabs = ((x0r, pkw, owa, owb), (x2r, pkb, oba, obb), (x3r, pkt, ota, otb))
    for xr, tab, oa, ob in tabs:
        pltpu.sync_copy(xr.at[w], idx_v)

        pltpu.sync_copy(zeros_hbm, bufa.at[0])
        pltpu.sync_copy(bufa.at[0], acca.at[pl.ds(s * _BPW, _BPW)])
        pltpu.sync_copy(bufa.at[0], accb.at[pl.ds(s * _BPW, _BPW)])

        def fire(j, p, tab=tab):
            pltpu.async_copy(tab.at[idx_v.at[j], pl.ds(0, _EA)],
                             bufa.at[p], sema.at[p])
            pltpu.async_copy(tab.at[idx_v.at[j], pl.ds(_EA, _EA)],
                             bufb.at[p], semb.at[p])

        def drain(j, p, tab=tab):
            pltpu.make_async_copy(
                tab.at[idx_v.at[j], pl.ds(0, _EA)],
                bufa.at[p], sema.at[p]).wait()
            pltpu.sync_copy(bufa.at[p], acca.at[dest_v.at[j]], add=True)
            pltpu.make_async_copy(
                tab.at[idx_v.at[j], pl.ds(_EA, _EA)],
                bufb.at[p], semb.at[p]).wait()
            pltpu.sync_copy(bufb.at[p], accb.at[dest_v.at[j]], add=True)

        fire(0, 0)

        def body(j, carry, tab=tab):
            fire(j, lax.rem(j, 2), tab)
            drain(j - 1, lax.rem(j - 1, 2), tab)
            return carry

        lax.fori_loop(1, _NCHUNK, body, 0)
        drain(_NCHUNK - 1, (_NCHUNK - 1) % 2)

        pltpu.sync_copy(acca.at[pl.ds(s * _BPW, _BPW)], bufa.at[0])
        pltpu.sync_copy(bufa.at[0], oa.at[pl.ds(base, _BPW)])
        pltpu.sync_copy(accb.at[pl.ds(s * _BPW, _BPW)], bufb.at[0])
        pltpu.sync_copy(bufb.at[0], ob.at[pl.ds(base, _BPW)])


@jax.jit
def _sc_pool(x0r, x2r, x3r, dest, zeros, pkw, pkb, pkt):
    mesh = plsc.VectorSubcoreMesh(core_axis_name="c", subcore_axis_name="s")
    so = jax.ShapeDtypeStruct((_B, _EA), jnp.float32)
    return pl.kernel(
        _sc_pool_body,
        out_type=(so,) * 6,
        mesh=mesh,
        scratch_types=[
            pltpu.VMEM((_NCHUNK, _K), jnp.int32),
            pltpu.VMEM((_NCHUNK, _K), jnp.int32),
            pltpu.VMEM((2, _K, _EA), jnp.float32),
            pltpu.VMEM((2, _K, _EA), jnp.float32),
            pltpu.VMEM_SHARED((_NS * _BPW, _EA), jnp.float32),
            pltpu.VMEM_SHARED((_NS * _BPW, _EA), jnp.float32),
            pltpu.SemaphoreType.DMA((2,)),
            pltpu.SemaphoreType.DMA((2,)),
        ],
        compiler_params=pltpu.CompilerParams(use_tc_tiling_on_sc=True),
    )(x0r, x2r, x3r, dest, zeros, pkw, pkb, pkt)


def _mlp_body(pwa, pwb, pba, pbb, pta, ptb,
              w1wa, w1wb, w1ba, w1bb, w1ta, w1tb,
              b1r, w2r, b2r, out):
    h = jnp.dot(pwa[...], w1wa[...], preferred_element_type=jnp.float32)
    h += jnp.dot(pwb[...], w1wb[...], preferred_element_type=jnp.float32)
    h += jnp.dot(pba[...], w1ba[...], preferred_element_type=jnp.float32)
    h += jnp.dot(pbb[...], w1bb[...], preferred_element_type=jnp.float32)
    h += jnp.dot(pta[...], w1ta[...], preferred_element_type=jnp.float32)
    h += jnp.dot(ptb[...], w1tb[...], preferred_element_type=jnp.float32)
    h = h * (1.0 / _L) + b1r[...]
    h = jnp.maximum(h, 0.0)
    out[...] = jnp.dot(h, w2r[...], preferred_element_type=jnp.float32) + b2r[...]


_BB = 1024


@jax.jit
def _mlp(pwa, pwb, pba, pbb, pta, ptb, W1, b1, W2, b2):
    pad = jnp.zeros((_EA - (_E - _EA), _HID), jnp.float32)
    w1a = [W1[t * _E:t * _E + _EA] for t in range(3)]
    w1b_ = [jnp.concatenate([W1[t * _E + _EA:(t + 1) * _E], pad])
            for t in range(3)]
    grid = (_B // _BB,)
    blk = pl.BlockSpec((_BB, _EA), lambda i: (i, 0))
    full = lambda r, ccols: pl.BlockSpec((r, ccols), lambda i: (0, 0))
    return pl.pallas_call(
        _mlp_body,
        grid=grid,
        in_specs=[blk] * 6 + [full(_EA, _HID)] * 6
                 + [full(1, _HID), full(_HID, _NCLS), full(1, _NCLS)],
        out_specs=pl.BlockSpec((_BB, _NCLS), lambda i: (i, 0)),
        out_shape=jax.ShapeDtypeStruct((_B, _NCLS), jnp.float32),
    )(pwa, pwb, pba, pbb, pta, ptb,
      w1a[0], w1b_[0], w1a[1], w1b_[1], w1a[2], w1b_[2],
      b1.reshape(1, _HID), W2, b2.reshape(1, _NCLS))


def kernel(x0, x1, x2, x3, emb_word, emb_bigram, emb_trigram, W1, b1, W2, b2):
    del x1
    x0r = x0.reshape(_NW, _NCHUNK, _K)
    x2r = x2.reshape(_NW, _NCHUNK, _K)
    x3r = x3.reshape(_NW, _NCHUNK, _K)
    pos = (jnp.arange(_BPW * _L, dtype=jnp.int32) // _L).reshape(_NCHUNK, _K)
    dest = jnp.arange(_NS, dtype=jnp.int32)[:, None, None] * _BPW + pos[None]
    zeros = jnp.zeros((_K, _EA), jnp.float32)
    pkw, pkb, pkt = _pack(emb_word, emb_bigram, emb_trigram)
    pooled = _sc_pool(x0r, x2r, x3r, dest, zeros, pkw, pkb, pkt)
    return _mlp(*pooled, W1, b1, W2, b2)


# speedup vs baseline: 1.0539x; 1.0539x over previous
import functools

import jax
import jax.numpy as jnp
from jax import lax
from jax.experimental import pallas as pl
from jax.experimental.pallas import tpu as pltpu
from jax.experimental.pallas import tpu_sc as plsc

_B = 4096
_L = 50
_E = 200
_EP = 256
_HID = 256
_NCLS = 20

_NC = 2
_NS = 16
_NW = _NC * _NS
_BPW = _B // _NW
_K = 128
_NCHUNK = (_BPW * _L) // _K

_RV = 2048


def _tails(emb_word, emb_bigram, emb_trigram):
    return [jnp.pad(tab[:, _EA:_E], ((0, 0), (0, _EA - (_E - _EA))))
            for tab in (emb_word, emb_bigram, emb_trigram)]


_EA = 128


def _sc_pool_body(x0r, x2r, x3r, dest_hbm, zeros_hbm,
                  tw, tb, tt, tlw, tlb, tlt,
                  owa, owb, oba, obb, ota, otb,
                  idx_v, dest_v, bufa, bufb, acca, accb, sema, semb):
    c = lax.axis_index("c")
    s = lax.axis_index("s")
    w = c * _NS + s
    base = w * _BPW

    pltpu.sync_copy(dest_hbm.at[s], dest_v)

    tabs = ((x0r, tw, tlw, owa, owb),
            (x2r, tb, tlb, oba, obb),
            (x3r, tt, tlt, ota, otb))
    for xr, tab, tail, oa, ob in tabs:
        pltpu.sync_copy(xr.at[w], idx_v)

        pltpu.sync_copy(zeros_hbm, bufa.at[0])
        pltpu.sync_copy(bufa.at[0], acca.at[pl.ds(s * _BPW, _BPW)])
        pltpu.sync_copy(bufa.at[0], accb.at[pl.ds(s * _BPW, _BPW)])

        def fire(j, p, tab=tab, tail=tail):
            pltpu.async_copy(tab.at[idx_v.at[j], pl.ds(0, _EA)],
                             bufa.at[p], sema.at[p])
            pltpu.async_copy(tail.at[idx_v.at[j]], bufb.at[p], semb.at[p])

        def drain(j, p, tab=tab, tail=tail):
            pltpu.make_async_copy(
                tab.at[idx_v.at[j], pl.ds(0, _EA)],
                bufa.at[p], sema.at[p]).wait()
            pltpu.sync_copy(bufa.at[p], acca.at[dest_v.at[j]], add=True)
            pltpu.make_async_copy(
                tail.at[idx_v.at[j]], bufb.at[p], semb.at[p]).wait()
            pltpu.sync_copy(bufb.at[p], accb.at[dest_v.at[j]], add=True)

        fire(0, 0)

        def body(j, carry, tab=tab, tail=tail):
            fire(j, lax.rem(j, 2), tab, tail)
            drain(j - 1, lax.rem(j - 1, 2), tab, tail)
            return carry

        lax.fori_loop(1, _NCHUNK, body, 0)
        drain(_NCHUNK - 1, (_NCHUNK - 1) % 2)

        pltpu.sync_copy(acca.at[pl.ds(s * _BPW, _BPW)], bufa.at[0])
        pltpu.sync_copy(bufa.at[0], oa.at[pl.ds(base, _BPW)])
        pltpu.sync_copy(accb.at[pl.ds(s * _BPW, _BPW)], bufb.at[0])
        pltpu.sync_copy(bufb.at[0], ob.at[pl.ds(base, _BPW)])


@jax.jit
def _sc_pool(x0r, x2r, x3r, dest, zeros,
             emb_word, emb_bigram, emb_trigram):
    tail_w, tail_b, tail_t = _tails(emb_word, emb_bigram, emb_trigram)
    mesh = plsc.VectorSubcoreMesh(core_axis_name="c", subcore_axis_name="s")
    so = jax.ShapeDtypeStruct((_B, _EA), jnp.float32)
    return pl.kernel(
        _sc_pool_body,
        out_type=(so,) * 6,
        mesh=mesh,
        scratch_types=[
            pltpu.VMEM((_NCHUNK, _K), jnp.int32),
            pltpu.VMEM((_NCHUNK, _K), jnp.int32),
            pltpu.VMEM((2, _K, _EA), jnp.float32),
            pltpu.VMEM((2, _K, _EA), jnp.float32),
            pltpu.VMEM_SHARED((_NS * _BPW, _EA), jnp.float32),
            pltpu.VMEM_SHARED((_NS * _BPW, _EA), jnp.float32),
            pltpu.SemaphoreType.DMA((2,)),
            pltpu.SemaphoreType.DMA((2,)),
        ],
        compiler_params=pltpu.CompilerParams(use_tc_tiling_on_sc=True),
    )(x0r, x2r, x3r, dest, zeros,
      emb_word, emb_bigram, emb_trigram, tail_w, tail_b, tail_t)


def _mlp_body(pwa, pwb, pba, pbb, pta, ptb,
              w1wa, w1wb, w1ba, w1bb, w1ta, w1tb,
              b1r, w2r, b2r, out):
    h = jnp.dot(pwa[...], w1wa[...], preferred_element_type=jnp.float32)
    h += jnp.dot(pwb[...], w1wb[...], preferred_element_type=jnp.float32)
    h += jnp.dot(pba[...], w1ba[...], preferred_element_type=jnp.float32)
    h += jnp.dot(pbb[...], w1bb[...], preferred_element_type=jnp.float32)
    h += jnp.dot(pta[...], w1ta[...], preferred_element_type=jnp.float32)
    h += jnp.dot(ptb[...], w1tb[...], preferred_element_type=jnp.float32)
    h = h * (1.0 / _L) + b1r[...]
    h = jnp.maximum(h, 0.0)
    out[...] = jnp.dot(h, w2r[...], preferred_element_type=jnp.float32) + b2r[...]


_BB = 1024


@jax.jit
def _mlp(pwa, pwb, pba, pbb, pta, ptb, W1, b1, W2, b2):
    pad = jnp.zeros((_EA - (_E - _EA), _HID), jnp.float32)
    w1a = [W1[t * _E:t * _E + _EA] for t in range(3)]
    w1b_ = [jnp.concatenate([W1[t * _E + _EA:(t + 1) * _E], pad])
            for t in range(3)]
    grid = (_B // _BB,)
    blk = pl.BlockSpec((_BB, _EA), lambda i: (i, 0))
    full = lambda r, ccols: pl.BlockSpec((r, ccols), lambda i: (0, 0))
    return pl.pallas_call(
        _mlp_body,
        grid=grid,
        in_specs=[blk] * 6 + [full(_EA, _HID)] * 6
                 + [full(1, _HID), full(_HID, _NCLS), full(1, _NCLS)],
        out_specs=pl.BlockSpec((_BB, _NCLS), lambda i: (i, 0)),
        out_shape=jax.ShapeDtypeStruct((_B, _NCLS), jnp.float32),
    )(pwa, pwb, pba, pbb, pta, ptb,
      w1a[0], w1b_[0], w1a[1], w1b_[1], w1a[2], w1b_[2],
      b1.reshape(1, _HID), W2, b2.reshape(1, _NCLS))


def kernel(x0, x1, x2, x3, emb_word, emb_bigram, emb_trigram, W1, b1, W2, b2):
    del x1
    x0r = x0.reshape(_NW, _NCHUNK, _K)
    x2r = x2.reshape(_NW, _NCHUNK, _K)
    x3r = x3.reshape(_NW, _NCHUNK, _K)
    pos = (jnp.arange(_BPW * _L, dtype=jnp.int32) // _L).reshape(_NCHUNK, _K)
    dest = jnp.arange(_NS, dtype=jnp.int32)[:, None, None] * _BPW + pos[None]
    zeros = jnp.zeros((_K, _EA), jnp.float32)
    pooled = _sc_pool(x0r, x2r, x3r, dest, zeros,
                      emb_word, emb_bigram, emb_trigram)
    return _mlp(*pooled, W1, b1, W2, b2)
